# baseline (speedup 1.0000x reference)
import functools

import jax
import jax.numpy as jnp
from jax import lax
from jax.experimental import pallas as pl
from jax.experimental.pallas import tpu as pltpu
from jax.experimental.pallas import tpu_sc as plsc

_NUM_CORES = 2
_NUM_SUBCORES = 16
_NUM_WORKERS = _NUM_CORES * _NUM_SUBCORES


_TBLK = 4096


def _transpose_tc(table_t):
    d, v = table_t.shape
    grid = (v + _TBLK - 1) // _TBLK

    def body(x_ref, o_ref):
        o_ref[...] = x_ref[...].T

    return pl.pallas_call(
        body,
        grid=(grid,),
        in_specs=[pl.BlockSpec((d, _TBLK), lambda i: (0, i))],
        out_specs=pl.BlockSpec((_TBLK, d), lambda i: (i, 0)),
        out_shape=jax.ShapeDtypeStruct((v, d), jnp.float32),
    )(table_t)


@jax.jit
def _embed_gather(idx, table):
    b = idx.shape[0]
    d = table.shape[1]
    b_per_w = b // _NUM_WORKERS

    mesh = plsc.VectorSubcoreMesh(core_axis_name="c", subcore_axis_name="s")

    @functools.partial(
        pl.kernel,
        mesh=mesh,
        out_type=jax.ShapeDtypeStruct((b, d), jnp.float32),
        scratch_types=[
            pltpu.VMEM((b_per_w,), jnp.int32),
            pltpu.VMEM((b_per_w, d), jnp.float32),
            pltpu.SemaphoreType.DMA,
        ],
    )
    def k(idx_hbm, table_hbm, out_hbm, idx_v, rows_v, sem):
        wid = lax.axis_index("s") * _NUM_CORES + lax.axis_index("c")
        base = wid * b_per_w
        pltpu.sync_copy(idx_hbm.at[pl.ds(base, b_per_w)], idx_v)

        lanes = 16

        def issue(g, _):
            v = idx_v[pl.ds(g * lanes, lanes)]
            for lane in range(lanes):
                pltpu.async_copy(
                    table_hbm.at[pl.ds(v[lane], 1)],
                    rows_v.at[pl.ds(g * lanes + lane, 1)],
                    sem,
                )
            return 0

        lax.fori_loop(0, b_per_w // lanes, issue, 0)

        pltpu.make_async_copy(
            table_hbm.at[pl.ds(0, b_per_w)], rows_v, sem
        ).wait()

        pltpu.sync_copy(rows_v, out_hbm.at[pl.ds(base, b_per_w)])

    return k(idx, table)


def kernel(labels, embed_table):
    return _embed_gather(labels.astype(jnp.int32), embed_table)
